# baseline (speedup 1.0000x reference)
import functools

import jax
import jax.numpy as jnp
from jax import lax
from jax.experimental import pallas as pl
from jax.experimental.pallas import tpu as pltpu
from jax.experimental.pallas import tpu_sc as plsc

AXON = 1000
NEURON = 10000
T_STEPS = 16
BETA = 0.95
THRESH = 1.0
B = 128

NT = 1024
BT = 16

NC = 2
NS = 16
LANES = 16
NWORK = NC * NS
ROWS_PER_W = B // NWORK


def _matmul_body(x_ref, w_ref, m_ref, b_ref, cur_ref):
    wm = w_ref[...] * m_ref[...].astype(jnp.float32)
    cur_ref[...] = jax.lax.dot_general(
        x_ref[...], wm,
        dimension_numbers=(((1,), (1,)), ((), ())),
        preferred_element_type=jnp.float32,
    ) + b_ref[...]


def _spk_body(cur_ref, spk_ref):
    cur = cur_ref[...]
    mem = jnp.zeros_like(cur)
    for t in range(T_STEPS):
        reset = (mem > THRESH).astype(jnp.float32)
        mem = BETA * mem + cur - reset * THRESH
        spk_ref[t] = (mem > THRESH).astype(jnp.float32)


UNROLL = 5
VEC_ITERS = NEURON // (LANES * UNROLL)


def _sc_mem(cur):
    mesh = plsc.VectorSubcoreMesh(core_axis_name="c", subcore_axis_name="s")

    @functools.partial(
        pl.kernel,
        mesh=mesh,
        out_type=jax.ShapeDtypeStruct((T_STEPS, B, NEURON), jnp.float32),
        scratch_types=[
            pltpu.VMEM((ROWS_PER_W, NEURON), jnp.float32),
            pltpu.VMEM((ROWS_PER_W, NEURON), jnp.float32),
            pltpu.VMEM((ROWS_PER_W, NEURON), jnp.float32),
            pltpu.SemaphoreType.DMA,
            pltpu.SemaphoreType.DMA,
        ],
    )
    def k(cur_hbm, mem_hbm, cur_v, m_a, m_b, sem_a, sem_b):
        wid = lax.axis_index("s") * NC + lax.axis_index("c")
        base = wid * ROWS_PER_W
        pltpu.sync_copy(cur_hbm.at[pl.ds(base, ROWS_PER_W)], cur_v)
        bufs = (m_a, m_b)
        sems = (sem_a, sem_b)
        copies = [None, None]
        for t in range(T_STEPS):
            dst = bufs[t % 2]
            src = bufs[(t - 1) % 2]
            if copies[t % 2] is not None:
                copies[t % 2].wait()
            for r in range(ROWS_PER_W):
                def body(i, carry, r=r, t=t):
                    for u in range(UNROLL):
                        off = (i * UNROLL + u) * LANES
                        c = cur_v[r, pl.ds(off, LANES)]
                        if t == 0:
                            m = c
                        else:
                            m = src[r, pl.ds(off, LANES)]
                            reset = jnp.where(m > THRESH, THRESH, 0.0)
                            m = BETA * m + c - reset
                        dst[r, pl.ds(off, LANES)] = m
                    return carry
                lax.fori_loop(0, VEC_ITERS, body, 0)
            copies[t % 2] = pltpu.async_copy(
                dst, mem_hbm.at[t, pl.ds(base, ROWS_PER_W)], sems[t % 2])
        copies[0].wait()
        copies[1].wait()

    return k(cur)


def kernel(x, W, b, mask):
    b2 = b.reshape(1, NEURON)
    cur = pl.pallas_call(
        _matmul_body,
        grid=(pl.cdiv(NEURON, NT),),
        in_specs=[
            pl.BlockSpec((B, AXON), lambda i: (0, 0)),
            pl.BlockSpec((NT, AXON), lambda i: (i, 0)),
            pl.BlockSpec((NT, AXON), lambda i: (i, 0)),
            pl.BlockSpec((1, NT), lambda i: (0, i)),
        ],
        out_specs=pl.BlockSpec((B, NT), lambda i: (0, i)),
        out_shape=jax.ShapeDtypeStruct((B, NEURON), jnp.float32),
    )(x, W, mask, b2)

    mem = _sc_mem(cur)

    spk = pl.pallas_call(
        _spk_body,
        grid=(B // BT,),
        in_specs=[pl.BlockSpec((BT, NEURON), lambda i: (i, 0))],
        out_specs=pl.BlockSpec((T_STEPS, BT, NEURON), lambda i: (0, i, 0)),
        out_shape=jax.ShapeDtypeStruct((T_STEPS, B, NEURON), jnp.float32),
    )(cur)
    return spk, mem


# speedup vs baseline: 1.6751x; 1.6751x over previous
import jax
import jax.numpy as jnp
from jax.experimental import pallas as pl

AXON = 1000
NEURON = 10000
T_STEPS = 16
BETA = 0.95
THRESH = 1.0
B = 128

NT = 1024


def _lif_body(x_ref, w_ref, m8_ref, b_ref, spk_ref, mem_ref):
    m8 = m8_ref[...].astype(jnp.int32)
    e = jnp.broadcast_to(m8[:, None, :], (NT // 8, 8, AXON))
    e = e.reshape(NT, AXON)
    sh = 7 - (jax.lax.broadcasted_iota(jnp.int32, (NT, AXON), 0) % 8)
    bits = jnp.right_shift(e, sh) & 1
    wm = w_ref[...] * bits.astype(jnp.float32)
    cur = jax.lax.dot_general(
        x_ref[...], wm,
        dimension_numbers=(((1,), (1,)), ((), ())),
        preferred_element_type=jnp.float32,
    ) + b_ref[...]
    mem = jnp.zeros_like(cur)
    for t in range(T_STEPS):
        reset = (mem > THRESH).astype(jnp.float32)
        mem = BETA * mem + cur - reset * THRESH
        spk_ref[t] = (mem > THRESH).astype(jnp.float32)
        mem_ref[t] = mem


def kernel(x, W, b, mask):
    b2 = b.reshape(1, NEURON)
    m8 = jnp.packbits(mask, axis=0)
    grid = (NEURON // NT + (NEURON % NT > 0),)
    spk, mem = pl.pallas_call(
        _lif_body,
        grid=grid,
        in_specs=[
            pl.BlockSpec((B, AXON), lambda i: (0, 0)),
            pl.BlockSpec((NT, AXON), lambda i: (i, 0)),
            pl.BlockSpec((NT // 8, AXON), lambda i: (i, 0)),
            pl.BlockSpec((1, NT), lambda i: (0, i)),
        ],
        out_specs=[
            pl.BlockSpec((T_STEPS, B, NT), lambda i: (0, 0, i)),
            pl.BlockSpec((T_STEPS, B, NT), lambda i: (0, 0, i)),
        ],
        out_shape=[
            jax.ShapeDtypeStruct((T_STEPS, B, NEURON), jnp.float32),
            jax.ShapeDtypeStruct((T_STEPS, B, NEURON), jnp.float32),
        ],
    )(x, W, m8, b2)
    return spk, mem


# speedup vs baseline: 1.6755x; 1.0002x over previous
import jax
import jax.numpy as jnp
from jax.experimental import pallas as pl

AXON = 1000
NEURON = 10000
T_STEPS = 16
BETA = 0.95
THRESH = 1.0
B = 128

NT = 1280


def _lif_body(x_ref, w_ref, m8_ref, b_ref, spk_ref, mem_ref):
    m8 = m8_ref[...].astype(jnp.int32)
    e = jnp.broadcast_to(m8[:, None, :], (NT // 8, 8, AXON))
    e = e.reshape(NT, AXON)
    sh = 7 - (jax.lax.broadcasted_iota(jnp.int32, (NT, AXON), 0) % 8)
    bits = jnp.right_shift(e, sh) & 1
    wm = w_ref[...] * bits.astype(jnp.float32)
    cur = jax.lax.dot_general(
        x_ref[...], wm,
        dimension_numbers=(((1,), (1,)), ((), ())),
        preferred_element_type=jnp.float32,
    ) + b_ref[...]
    mem = jnp.zeros_like(cur)
    for t in range(T_STEPS):
        reset = (mem > THRESH).astype(jnp.float32)
        mem = BETA * mem + cur - reset * THRESH
        spk_ref[t] = (mem > THRESH).astype(jnp.float32)
        mem_ref[t] = mem


def kernel(x, W, b, mask):
    b2 = b.reshape(1, NEURON)
    m8 = jnp.packbits(mask, axis=0)
    grid = (NEURON // NT + (NEURON % NT > 0),)
    spk, mem = pl.pallas_call(
        _lif_body,
        grid=grid,
        in_specs=[
            pl.BlockSpec((B, AXON), lambda i: (0, 0)),
            pl.BlockSpec((NT, AXON), lambda i: (i, 0)),
            pl.BlockSpec((NT // 8, AXON), lambda i: (i, 0)),
            pl.BlockSpec((1, NT), lambda i: (0, i)),
        ],
        out_specs=[
            pl.BlockSpec((T_STEPS, B, NT), lambda i: (0, 0, i)),
            pl.BlockSpec((T_STEPS, B, NT), lambda i: (0, 0, i)),
        ],
        out_shape=[
            jax.ShapeDtypeStruct((T_STEPS, B, NEURON), jnp.float32),
            jax.ShapeDtypeStruct((T_STEPS, B, NEURON), jnp.float32),
        ],
    )(x, W, m8, b2)
    return spk, mem
